# baseline (speedup 1.0000x reference)
import jax
import jax.numpy as jnp
from jax.experimental import pallas as pl
from jax.experimental.pallas import tpu as pltpu

_M = 4096
_N = 4096
_BG = 128
_BM = _BG * 8


def _loss_body(A_ref, xb_ref, b_ref, Iy_ref, x_ref, il_ref, iu_ref, l_ref,
               u_ref, out_ref, acc_ref):
    i = pl.program_id(0)
    nb = pl.num_programs(0)

    @pl.when(i == 0)
    def _init():
        xv = x_ref[...]
        vv = (jnp.maximum(l_ref[...] - xv, 0.0) * il_ref[...]
              + jnp.maximum(xv - u_ref[...], 0.0) * iu_ref[...])
        bv = b_ref[...]
        acc_ref[0] = jnp.sum(vv * vv)
        acc_ref[1] = jnp.sum(bv * bv)
        acc_ref[2] = 0.0

    ax = jnp.sum(A_ref[...] * xb_ref[...][None], axis=2)
    bb = b_ref[pl.ds(i * _BG, _BG), :]
    cv = bb - ax
    cv = cv + jnp.maximum(-cv, 0.0) * Iy_ref[pl.ds(i * _BG, _BG), :]
    acc_ref[2] += jnp.sum(cv * cv)

    @pl.when(i == nb - 1)
    def _fin():
        part_2 = jnp.sqrt(acc_ref[0] + acc_ref[2])
        part_3 = 1.0 + jnp.sqrt(acc_ref[1])
        out_ref[0] = part_2 / part_3


def kernel(A, b, c, x, Iy, il, iu, l, u):
    del c
    A3 = A.reshape(_M // 8, 8, _N)
    xb = jnp.broadcast_to(x.reshape(1, _N), (8, _N))
    b8 = b.reshape(_M // 8, 8)
    Iy8 = Iy.reshape(_M // 8, 8)
    small = [v.reshape(32, 128) for v in (x, il, iu, l, u)]
    full8 = pl.BlockSpec((_M // 8, 8), lambda i: (0, 0))
    full = pl.BlockSpec((32, 128), lambda i: (0, 0))
    out = pl.pallas_call(
        _loss_body,
        grid=(_M // _BM,),
        in_specs=[
            pl.BlockSpec((_BG, 8, _N), lambda i: (i, 0, 0)),
            pl.BlockSpec((8, _N), lambda i: (0, 0)),
            full8,
            full8,
            full,
            full,
            full,
            full,
            full,
        ],
        out_specs=pl.BlockSpec(memory_space=pltpu.SMEM),
        out_shape=jax.ShapeDtypeStruct((1,), jnp.float32),
        scratch_shapes=[pltpu.SMEM((3,), jnp.float32)],
    )(A3, xb, b8, Iy8, *small)
    return out[0]


# speedup vs baseline: 1.0740x; 1.0740x over previous
import jax
import jax.numpy as jnp
from jax.experimental import pallas as pl
from jax.experimental.pallas import tpu as pltpu

_M = 4096
_N = 4096
_G = _M // 8
_BG = 32
_NSTEP = _G // (2 * _BG)


def _loss_body(A0_ref, A1_ref, xb_ref, b_ref, Iy_ref, x_ref, il_ref, iu_ref,
               l_ref, u_ref, out_ref, acc_ref):
    i = pl.program_id(0)
    nb = pl.num_programs(0)

    @pl.when(i == 0)
    def _init():
        xv = x_ref[...]
        vv = (jnp.maximum(l_ref[...] - xv, 0.0) * il_ref[...]
              + jnp.maximum(xv - u_ref[...], 0.0) * iu_ref[...])
        bv = b_ref[...]
        acc_ref[0] = jnp.sum(vv * vv)
        acc_ref[1] = jnp.sum(bv * bv)
        acc_ref[2] = 0.0

    xb = xb_ref[...][None]
    total = 0.0
    for half, a_ref in enumerate((A0_ref, A1_ref)):
        base = half * (_G // 2) + i * _BG
        ax = jnp.sum(a_ref[...] * xb, axis=2)
        bb = b_ref[pl.ds(base, _BG), :]
        cv = bb - ax
        cv = cv + jnp.maximum(-cv, 0.0) * Iy_ref[pl.ds(base, _BG), :]
        total = total + jnp.sum(cv * cv)
    acc_ref[2] += total

    @pl.when(i == nb - 1)
    def _fin():
        part_2 = jnp.sqrt(acc_ref[0] + acc_ref[2])
        part_3 = 1.0 + jnp.sqrt(acc_ref[1])
        out_ref[0] = part_2 / part_3


def kernel(A, b, c, x, Iy, il, iu, l, u):
    del c
    A3 = A.reshape(_G, 8, _N)
    xb = jnp.broadcast_to(x.reshape(1, _N), (8, _N))
    b8 = b.reshape(_G, 8)
    Iy8 = Iy.reshape(_G, 8)
    small = [v.reshape(32, 128) for v in (x, il, iu, l, u)]
    full8 = pl.BlockSpec((_G, 8), lambda i: (0, 0))
    full = pl.BlockSpec((32, 128), lambda i: (0, 0))
    out = pl.pallas_call(
        _loss_body,
        grid=(_NSTEP,),
        in_specs=[
            pl.BlockSpec((_BG, 8, _N), lambda i: (i, 0, 0)),
            pl.BlockSpec((_BG, 8, _N),
                         lambda i: (i + _G // (2 * _BG), 0, 0)),
            pl.BlockSpec((8, _N), lambda i: (0, 0)),
            full8,
            full8,
            full,
            full,
            full,
            full,
            full,
        ],
        out_specs=pl.BlockSpec(memory_space=pltpu.SMEM),
        out_shape=jax.ShapeDtypeStruct((1,), jnp.float32),
        scratch_shapes=[pltpu.SMEM((3,), jnp.float32)],
    )(A3, A3, xb, b8, Iy8, *small)
    return out[0]
